# baseline (speedup 1.0000x reference)
import functools

import jax
import jax.numpy as jnp
from jax.experimental import pallas as pl
from jax.experimental.pallas import tpu as pltpu

B = 4096
D = 128
X = 100000
XPAD = 100352
BT = 256
XT = 2048
NT = XPAD // XT
G1 = 16
NB1 = XPAD // G1
NB1P = 56 * 128
G2 = 8
NB2 = NB1P // G2
K = 100
NEG = -jnp.inf
IBIG = 2**30


def _matmul_block(q_ref, e_ref, s_ref, m1_ref):
    j = pl.program_id(1)
    s = jnp.dot(q_ref[...], e_ref[...], preferred_element_type=jnp.float32)
    col = j * XT + jax.lax.broadcasted_iota(jnp.int32, (BT, XT), 1)
    s = jnp.where(col < X, s, NEG)
    s_ref[...] = s
    m1_ref[...] = jnp.max(s.reshape(BT, G1, 128), axis=1)


def _pop_body(p, carry, pay, lane):
    v, accv, acci = carry
    m = jnp.max(v, axis=1, keepdims=True)
    sel = v == m
    pid = jnp.min(jnp.where(sel, pay, IBIG), axis=1, keepdims=True)
    v = jnp.where(pay == pid, NEG, v)
    accv = jnp.where(lane == p, m, accv)
    acci = jnp.where(lane == p, pid, acci)
    return v, accv, acci


def _run_pops(v, pay, pops, vout_ref, iout_ref):
    lane = jax.lax.broadcasted_iota(jnp.int32, (BT, 128), 1)
    accv0 = jnp.full((BT, 128), NEG, jnp.float32)
    acci0 = jnp.full((BT, 128), -1, jnp.int32)
    body = functools.partial(_pop_body, pay=pay, lane=lane)
    _, accv, acci = jax.lax.fori_loop(0, pops, lambda p, c: body(p, c), (v, accv0, acci0))
    vout_ref[...] = accv
    iout_ref[...] = acci


def _popk_block(v_ref, p_ref, vout_ref, iout_ref, *, pops):
    _run_pops(v_ref[...], p_ref[...], pops, vout_ref, iout_ref)


def _popk(vals, payload, width, pops):
    return pl.pallas_call(
        functools.partial(_popk_block, pops=pops),
        grid=(B // BT,),
        in_specs=[
            pl.BlockSpec((BT, width), lambda i: (i, 0)),
            pl.BlockSpec((BT, width), lambda i: (i, 0)),
        ],
        out_specs=[
            pl.BlockSpec((BT, 128), lambda i: (i, 0)),
            pl.BlockSpec((BT, 128), lambda i: (i, 0)),
        ],
        out_shape=[
            jax.ShapeDtypeStruct((B, 128), jnp.float32),
            jax.ShapeDtypeStruct((B, 128), jnp.int32),
        ],
    )(vals, payload)


def _pop2_block(m1_ref, vout_ref, iout_ref, *, pops):
    m1 = m1_ref[...]
    m1p = jnp.concatenate(
        [m1, jnp.full((BT, NB1P - NB1), NEG, jnp.float32)], axis=1)
    m2 = jnp.max(m1p.reshape(BT, G2, NB2), axis=1)
    m2 = jnp.concatenate(
        [m2, jnp.full((BT, 1024 - NB2), NEG, jnp.float32)], axis=1)
    pay = jax.lax.broadcasted_iota(jnp.int32, (BT, 1024), 1)
    _run_pops(m2, pay, pops, vout_ref, iout_ref)


def kernel(query_embeddings, item_embeddings_t, ids, k):
    e_pad = jnp.pad(item_embeddings_t, ((0, 0), (0, XPAD - X)))
    scores, m1 = pl.pallas_call(
        _matmul_block,
        grid=(B // BT, NT),
        in_specs=[
            pl.BlockSpec((BT, D), lambda i, j: (i, 0)),
            pl.BlockSpec((D, XT), lambda i, j: (0, j)),
        ],
        out_specs=[
            pl.BlockSpec((BT, XT), lambda i, j: (i, j)),
            pl.BlockSpec((BT, 128), lambda i, j: (i, j)),
        ],
        out_shape=[
            jax.ShapeDtypeStruct((B, XPAD), jnp.float32),
            jax.ShapeDtypeStruct((B, NB1), jnp.float32),
        ],
        compiler_params=pltpu.CompilerParams(
            dimension_semantics=("parallel", "arbitrary"),
        ),
    )(query_embeddings, e_pad)

    _, bids2 = pl.pallas_call(
        functools.partial(_pop2_block, pops=K),
        grid=(B // BT,),
        in_specs=[pl.BlockSpec((BT, NB1), lambda i: (i, 0))],
        out_specs=[
            pl.BlockSpec((BT, 128), lambda i: (i, 0)),
            pl.BlockSpec((BT, 128), lambda i: (i, 0)),
        ],
        out_shape=[
            jax.ShapeDtypeStruct((B, 128), jnp.float32),
            jax.ShapeDtypeStruct((B, 128), jnp.int32),
        ],
    )(m1)
    bids2 = bids2[:, :K]

    m1p = jnp.pad(m1, ((0, 0), (0, NB1P - NB1)), constant_values=NEG)
    i1 = (bids2[:, :, None]
          + (NB2 * jnp.arange(G2, dtype=jnp.int32))[None, None, :]).reshape(B, K * G2)
    c1 = jnp.take_along_axis(m1p, i1, axis=1)
    c1 = jnp.concatenate(
        [c1, jnp.full((B, 1024 - K * G2), NEG, jnp.float32)], axis=1)
    i1 = jnp.concatenate(
        [i1, jnp.full((B, 1024 - K * G2), IBIG, jnp.int32)], axis=1)

    _, bids1 = _popk(c1, i1, 1024, K)
    bids1 = bids1[:, :K]

    base = (bids1 // 128) * XT + (bids1 % 128)
    i0 = (base[:, :, None]
          + (128 * jnp.arange(G1, dtype=jnp.int32))[None, None, :]).reshape(B, K * G1)
    c0 = jnp.take_along_axis(scores, i0, axis=1)
    c0 = jnp.concatenate(
        [c0, jnp.full((B, 1664 - K * G1), NEG, jnp.float32)], axis=1)
    i0 = jnp.concatenate(
        [i0, jnp.full((B, 1664 - K * G1), IBIG, jnp.int32)], axis=1)

    svals, scols = _popk(c0, i0, 1664, K)
    top_scores = svals[:, :K]
    top_cols = scols[:, :K]
    top_ids = jnp.take_along_axis(
        ids, top_cols.reshape(1, B * K), axis=1).reshape(B, K)
    return top_ids, top_scores


# speedup vs baseline: 1.0007x; 1.0007x over previous
import functools

import jax
import jax.numpy as jnp
from jax.experimental import pallas as pl
from jax.experimental.pallas import tpu as pltpu

B = 4096
D = 128
X = 100000
XPAD = 100352
BT = 256
XT = 2048
NT = XPAD // XT
G1 = 16
NB1 = XPAD // G1
NB1P = 56 * 128
G2 = 8
NB2 = NB1P // G2
K = 100
NEG = -jnp.inf
IBIG = 2**30


def _matmul_block(q_ref, e_ref, s_ref, m1_ref):
    j = pl.program_id(1)
    s = jnp.dot(q_ref[...], e_ref[...], preferred_element_type=jnp.float32)
    col = j * XT + jax.lax.broadcasted_iota(jnp.int32, (BT, XT), 1)
    s = jnp.where(col < X, s, NEG)
    s_ref[...] = s
    m1_ref[...] = jnp.max(s.reshape(BT, G1, 128), axis=1)


def _pop_body(p, carry, pay, lane):
    v, accv, acci = carry
    m = jnp.max(v, axis=1, keepdims=True)
    sel = v == m
    pid = jnp.min(jnp.where(sel, pay, IBIG), axis=1, keepdims=True)
    v = jnp.where(pay == pid, NEG, v)
    accv = jnp.where(lane == p, m, accv)
    acci = jnp.where(lane == p, pid, acci)
    return v, accv, acci


def _run_pops(v, pay, pops, vout_ref, iout_ref):
    lane = jax.lax.broadcasted_iota(jnp.int32, (BT, 128), 1)
    accv0 = jnp.full((BT, 128), NEG, jnp.float32)
    acci0 = jnp.full((BT, 128), -1, jnp.int32)
    body = functools.partial(_pop_body, pay=pay, lane=lane)
    _, accv, acci = jax.lax.fori_loop(0, pops, lambda p, c: body(p, c), (v, accv0, acci0))
    vout_ref[...] = accv
    iout_ref[...] = acci


def _popk_block(v_ref, p_ref, vout_ref, iout_ref, *, pops):
    _run_pops(v_ref[...], p_ref[...], pops, vout_ref, iout_ref)


def _popk(vals, payload, width, pops):
    return pl.pallas_call(
        functools.partial(_popk_block, pops=pops),
        grid=(B // BT,),
        in_specs=[
            pl.BlockSpec((BT, width), lambda i: (i, 0)),
            pl.BlockSpec((BT, width), lambda i: (i, 0)),
        ],
        out_specs=[
            pl.BlockSpec((BT, 128), lambda i: (i, 0)),
            pl.BlockSpec((BT, 128), lambda i: (i, 0)),
        ],
        out_shape=[
            jax.ShapeDtypeStruct((B, 128), jnp.float32),
            jax.ShapeDtypeStruct((B, 128), jnp.int32),
        ],
    )(vals, payload)


def _pop2_block(m1_ref, vout_ref, iout_ref, *, pops):
    m1 = m1_ref[...]
    m1p = jnp.concatenate(
        [m1, jnp.full((BT, NB1P - NB1), NEG, jnp.float32)], axis=1)
    m2 = jnp.max(m1p.reshape(BT, G2, NB2), axis=1)
    m2 = jnp.concatenate(
        [m2, jnp.full((BT, 1024 - NB2), NEG, jnp.float32)], axis=1)
    pay = jax.lax.broadcasted_iota(jnp.int32, (BT, 1024), 1)
    _run_pops(m2, pay, pops, vout_ref, iout_ref)


def kernel(query_embeddings, item_embeddings_t, ids, k):
    e_pad = jnp.pad(item_embeddings_t, ((0, 0), (0, XPAD - X)))
    scores, m1 = pl.pallas_call(
        _matmul_block,
        grid=(B // BT, NT),
        in_specs=[
            pl.BlockSpec((BT, D), lambda i, j: (i, 0)),
            pl.BlockSpec((D, XT), lambda i, j: (0, j)),
        ],
        out_specs=[
            pl.BlockSpec((BT, XT), lambda i, j: (i, j)),
            pl.BlockSpec((BT, 128), lambda i, j: (i, j)),
        ],
        out_shape=[
            jax.ShapeDtypeStruct((B, XPAD), jnp.float32),
            jax.ShapeDtypeStruct((B, NB1), jnp.float32),
        ],
        compiler_params=pltpu.CompilerParams(
            dimension_semantics=("parallel", "arbitrary"),
        ),
    )(query_embeddings, e_pad)

    _, bids2 = pl.pallas_call(
        functools.partial(_pop2_block, pops=K),
        grid=(B // BT,),
        in_specs=[pl.BlockSpec((BT, NB1), lambda i: (i, 0))],
        out_specs=[
            pl.BlockSpec((BT, 128), lambda i: (i, 0)),
            pl.BlockSpec((BT, 128), lambda i: (i, 0)),
        ],
        out_shape=[
            jax.ShapeDtypeStruct((B, 128), jnp.float32),
            jax.ShapeDtypeStruct((B, 128), jnp.int32),
        ],
    )(m1)
    bids2 = bids2[:, :K]

    m1p = jnp.pad(m1, ((0, 0), (0, NB1P - NB1)), constant_values=NEG)
    i1 = (bids2[:, :, None]
          + (NB2 * jnp.arange(G2, dtype=jnp.int32))[None, None, :]).reshape(B, K * G2)
    c1 = jnp.take_along_axis(m1p, i1, axis=1)
    c1 = jnp.concatenate(
        [c1, jnp.full((B, 1024 - K * G2), NEG, jnp.float32)], axis=1)
    i1 = jnp.concatenate(
        [i1, jnp.full((B, 1024 - K * G2), IBIG, jnp.int32)], axis=1)

    _, bids1 = _popk(c1, i1, 1024, K)
    bids1 = bids1[:, :K]

    base = (bids1 // 128) * XT + (bids1 % 128)
    i0 = (base[:, :, None]
          + (128 * jnp.arange(G1, dtype=jnp.int32))[None, None, :]).reshape(B, K * G1)
    c0 = jnp.take_along_axis(scores, i0, axis=1)
    c0 = jnp.concatenate(
        [c0, jnp.full((B, 1664 - K * G1), NEG, jnp.float32)], axis=1)
    i0 = jnp.concatenate(
        [i0, jnp.full((B, 1664 - K * G1), IBIG, jnp.int32)], axis=1)

    svals, scols = _popk(c0, i0, 1664, K)
    top_scores = svals[:, :K]
    top_cols = scols[:, :K]
    top_ids = jnp.take(
        ids.reshape(X, 1), top_cols.reshape(B * K), axis=0).reshape(B, K)
    return top_ids, top_scores


# speedup vs baseline: 1.7442x; 1.7431x over previous
import functools

import jax
import jax.numpy as jnp
from jax import lax
from jax.experimental import pallas as pl
from jax.experimental.pallas import tpu as pltpu
from jax.experimental.pallas import tpu_sc as plsc

B = 4096
D = 128
X = 100000
XPAD = 100352
BT = 256
XT = 2048
NT = XPAD // XT
G1 = 16
NB1 = XPAD // G1
NB1P = 56 * 128
G2 = 8
NB2 = NB1P // G2
K = 100
NEG = -jnp.inf
IBIG = 2**30


def _matmul_block(q_ref, e_ref, s_ref, m1_ref):
    j = pl.program_id(1)
    s = jnp.dot(q_ref[...], e_ref[...], preferred_element_type=jnp.float32)
    col = j * XT + jax.lax.broadcasted_iota(jnp.int32, (BT, XT), 1)
    s = jnp.where(col < X, s, NEG)
    s_ref[...] = s
    m1_ref[...] = jnp.max(s.reshape(BT, G1, 128), axis=1)


def _pop_body(p, carry, pay, lane):
    v, accv, acci = carry
    m = jnp.max(v, axis=1, keepdims=True)
    sel = v == m
    pid = jnp.min(jnp.where(sel, pay, IBIG), axis=1, keepdims=True)
    v = jnp.where(pay == pid, NEG, v)
    accv = jnp.where(lane == p, m, accv)
    acci = jnp.where(lane == p, pid, acci)
    return v, accv, acci


def _run_pops(v, pay, pops, vout_ref, iout_ref):
    lane = jax.lax.broadcasted_iota(jnp.int32, (BT, 128), 1)
    accv0 = jnp.full((BT, 128), NEG, jnp.float32)
    acci0 = jnp.full((BT, 128), -1, jnp.int32)
    body = functools.partial(_pop_body, pay=pay, lane=lane)
    _, accv, acci = jax.lax.fori_loop(0, pops, lambda p, c: body(p, c), (v, accv0, acci0))
    vout_ref[...] = accv
    iout_ref[...] = acci


def _popk_block(v_ref, p_ref, vout_ref, iout_ref, *, pops):
    _run_pops(v_ref[...], p_ref[...], pops, vout_ref, iout_ref)


def _popk(vals, payload, width, pops):
    return pl.pallas_call(
        functools.partial(_popk_block, pops=pops),
        grid=(B // BT,),
        in_specs=[
            pl.BlockSpec((BT, width), lambda i: (i, 0)),
            pl.BlockSpec((BT, width), lambda i: (i, 0)),
        ],
        out_specs=[
            pl.BlockSpec((BT, 128), lambda i: (i, 0)),
            pl.BlockSpec((BT, 128), lambda i: (i, 0)),
        ],
        out_shape=[
            jax.ShapeDtypeStruct((B, 128), jnp.float32),
            jax.ShapeDtypeStruct((B, 128), jnp.int32),
        ],
    )(vals, payload)


def _pop2_block(m1_ref, vout_ref, iout_ref, *, pops):
    m1 = m1_ref[...]
    m1p = jnp.concatenate(
        [m1, jnp.full((BT, NB1P - NB1), NEG, jnp.float32)], axis=1)
    m2 = jnp.max(m1p.reshape(BT, G2, NB2), axis=1)
    m2 = jnp.concatenate(
        [m2, jnp.full((BT, 1024 - NB2), NEG, jnp.float32)], axis=1)
    pay = jax.lax.broadcasted_iota(jnp.int32, (BT, 1024), 1)
    _run_pops(m2, pay, pops, vout_ref, iout_ref)


def kernel(query_embeddings, item_embeddings_t, ids, k):
    e_pad = jnp.pad(item_embeddings_t, ((0, 0), (0, XPAD - X)))
    scores, m1 = pl.pallas_call(
        _matmul_block,
        grid=(B // BT, NT),
        in_specs=[
            pl.BlockSpec((BT, D), lambda i, j: (i, 0)),
            pl.BlockSpec((D, XT), lambda i, j: (0, j)),
        ],
        out_specs=[
            pl.BlockSpec((BT, XT), lambda i, j: (i, j)),
            pl.BlockSpec((BT, 128), lambda i, j: (i, j)),
        ],
        out_shape=[
            jax.ShapeDtypeStruct((B, XPAD), jnp.float32),
            jax.ShapeDtypeStruct((B, NB1), jnp.float32),
        ],
        compiler_params=pltpu.CompilerParams(
            dimension_semantics=("parallel", "arbitrary"),
        ),
    )(query_embeddings, e_pad)

    _, bids2 = pl.pallas_call(
        functools.partial(_pop2_block, pops=K),
        grid=(B // BT,),
        in_specs=[pl.BlockSpec((BT, NB1), lambda i: (i, 0))],
        out_specs=[
            pl.BlockSpec((BT, 128), lambda i: (i, 0)),
            pl.BlockSpec((BT, 128), lambda i: (i, 0)),
        ],
        out_shape=[
            jax.ShapeDtypeStruct((B, 128), jnp.float32),
            jax.ShapeDtypeStruct((B, 128), jnp.int32),
        ],
    )(m1)
    bids2 = bids2[:, :K]

    m1p = jnp.pad(m1, ((0, 0), (0, NB1P - NB1)), constant_values=NEG)
    i1 = (bids2[:, :, None]
          + (NB2 * jnp.arange(G2, dtype=jnp.int32))[None, None, :]).reshape(B, K * G2)
    c1 = jnp.take_along_axis(m1p, i1, axis=1)
    c1 = jnp.concatenate(
        [c1, jnp.full((B, 1024 - K * G2), NEG, jnp.float32)], axis=1)
    i1 = jnp.concatenate(
        [i1, jnp.full((B, 1024 - K * G2), IBIG, jnp.int32)], axis=1)

    _, bids1 = _popk(c1, i1, 1024, K)
    bids1 = bids1[:, :K]

    base = (bids1 // 128) * XT + (bids1 % 128)
    i0 = (base[:, :, None]
          + (128 * jnp.arange(G1, dtype=jnp.int32))[None, None, :]).reshape(B, K * G1)
    c0 = jnp.take_along_axis(scores, i0, axis=1)
    c0 = jnp.concatenate(
        [c0, jnp.full((B, 1664 - K * G1), NEG, jnp.float32)], axis=1)
    i0 = jnp.concatenate(
        [i0, jnp.full((B, 1664 - K * G1), IBIG, jnp.int32)], axis=1)

    svals, scols = _popk(c0, i0, 1664, K)
    top_scores = svals[:, :K]
    top_cols = scols[:, :K]
    tab = jnp.pad(ids, ((0, 0), (0, XPAD - X))).reshape(XPAD // 128, 128)
    rows = _sc_row_gather(tab, (top_cols // 128).reshape(B * K))
    lane_sel = (jnp.arange(128, dtype=jnp.int32)[None, None, :]
                == (top_cols % 128)[:, :, None])
    top_ids = jnp.sum(jnp.where(lane_sel, rows.reshape(B, K, 128), 0),
                      axis=2, dtype=jnp.int32)
    return top_ids, top_scores


def _sc_row_gather(table, ridx):
    n = ridx.shape[0]
    info = plsc.get_sparse_core_info()
    nw = info.num_cores * info.num_subcores
    b_per_w = n // nw
    assert b_per_w * nw == n
    chunk = 800
    n_chunks = b_per_w // chunk
    assert n_chunks * chunk == b_per_w
    mesh = plsc.VectorSubcoreMesh(core_axis_name="c", subcore_axis_name="s")

    @functools.partial(
        pl.kernel, mesh=mesh,
        out_type=jax.ShapeDtypeStruct((n, 128), jnp.int32),
        scratch_types=[
            pltpu.VMEM((chunk,), jnp.int32),
            pltpu.VMEM((chunk, 128), jnp.int32),
            pltpu.SemaphoreType.DMA,
        ],
    )
    def k(table_hbm, idx_hbm, out_hbm, idx_v, rows_v, sem):
        wid = lax.axis_index("s") * info.num_cores + lax.axis_index("c")
        for c in range(n_chunks):
            base = wid * b_per_w + c * chunk
            pltpu.sync_copy(idx_hbm.at[pl.ds(base, chunk)], idx_v)
            pltpu.async_copy(table_hbm.at[idx_v], rows_v, sem).wait()
            pltpu.sync_copy(rows_v, out_hbm.at[pl.ds(base, chunk)])

    return k(table, ridx)
